# Initial kernel scaffold; baseline (speedup 1.0000x reference)
import jax
import jax.numpy as jnp
from jax.experimental import pallas as pl


def kernel(x, centroids):
    raise NotImplementedError("write your pallas kernel here")


# speedup vs baseline: 2.8633x; 2.8633x over previous
import functools

import jax
import jax.numpy as jnp
from jax.experimental import pallas as pl


def _labels_kernel(x_ref, cent_ref, out_ref):
    xb = x_ref[0]
    cent = cent_ref[...]
    xy = jax.lax.dot_general(
        cent, xb, (((0,), (0,)), ((), ())),
        preferred_element_type=jnp.float32)
    xsq = jnp.sum(xb * xb, axis=0, keepdims=True)
    csq = jnp.sum(cent * cent, axis=0)[:, None]
    dist = xsq - 2.0 * xy + csq
    out_ref[0] = jnp.argmin(dist, axis=0).astype(jnp.int32)[None, :]


def kernel(x, centroids):
    b, c, s = x.shape
    k = centroids.shape[1]
    out = pl.pallas_call(
        _labels_kernel,
        grid=(b,),
        in_specs=[
            pl.BlockSpec((1, c, s), lambda i: (i, 0, 0)),
            pl.BlockSpec((c, k), lambda i: (0, 0)),
        ],
        out_specs=pl.BlockSpec((1, 1, s), lambda i: (i, 0, 0)),
        out_shape=jax.ShapeDtypeStruct((b, 1, s), jnp.int32),
    )(x, centroids)
    return out.reshape(b, s)


# speedup vs baseline: 3.1028x; 1.0837x over previous
import functools

import jax
import jax.numpy as jnp
from jax.experimental import pallas as pl


def _labels_kernel(x_ref, cent_ref, out_ref):
    xb = x_ref[0]
    cent = cent_ref[...]
    xyneg = jax.lax.dot_general(
        cent * -2.0, xb, (((0,), (0,)), ((), ())),
        preferred_element_type=jnp.float32)
    xsq = jnp.sum(xb * xb, axis=0, keepdims=True)
    csq = jnp.sum(cent * cent, axis=0)[:, None]
    dist = (xsq + xyneg) + csq
    out_ref[0] = jnp.argmin(dist, axis=0).astype(jnp.int32)[None, :]


def kernel(x, centroids):
    b, c, s = x.shape
    k = centroids.shape[1]
    out = pl.pallas_call(
        _labels_kernel,
        grid=(b,),
        in_specs=[
            pl.BlockSpec((1, c, s), lambda i: (i, 0, 0)),
            pl.BlockSpec((c, k), lambda i: (0, 0)),
        ],
        out_specs=pl.BlockSpec((1, 1, s), lambda i: (i, 0, 0)),
        out_shape=jax.ShapeDtypeStruct((b, 1, s), jnp.int32),
    )(x, centroids)
    return out.reshape(b, s)


# speedup vs baseline: 3.3192x; 1.0698x over previous
import functools

import jax
import jax.numpy as jnp
from jax.experimental import pallas as pl


_BB = 4


def _labels_kernel(x_ref, cent_ref, out_ref):
    cent = cent_ref[...]
    cneg = cent * -2.0
    csq = jnp.sum(cent * cent, axis=0)[:, None]
    for i in range(_BB):
        xb = x_ref[i]
        xyneg = jax.lax.dot_general(
            cneg, xb, (((0,), (0,)), ((), ())),
            preferred_element_type=jnp.float32)
        xsq = jnp.sum(xb * xb, axis=0, keepdims=True)
        dist = (xsq + xyneg) + csq
        out_ref[i] = jnp.argmin(dist, axis=0).astype(jnp.int32)[None, :]


def kernel(x, centroids):
    b, c, s = x.shape
    k = centroids.shape[1]
    out = pl.pallas_call(
        _labels_kernel,
        grid=(b // _BB,),
        in_specs=[
            pl.BlockSpec((_BB, c, s), lambda i: (i, 0, 0)),
            pl.BlockSpec((c, k), lambda i: (0, 0)),
        ],
        out_specs=pl.BlockSpec((_BB, 1, s), lambda i: (i, 0, 0)),
        out_shape=jax.ShapeDtypeStruct((b, 1, s), jnp.int32),
    )(x, centroids)
    return out.reshape(b, s)


# speedup vs baseline: 3.3929x; 1.0222x over previous
import functools

import jax
import jax.numpy as jnp
from jax.experimental import pallas as pl


_BB = 8


def _labels_kernel(x_ref, cent_ref, out_ref):
    cent = cent_ref[...]
    cneg = cent * -2.0
    csq = jnp.sum(cent * cent, axis=0)[:, None]
    for i in range(_BB):
        xb = x_ref[i]
        xyneg = jax.lax.dot_general(
            cneg, xb, (((0,), (0,)), ((), ())),
            preferred_element_type=jnp.float32)
        xsq = jnp.sum(xb * xb, axis=0, keepdims=True)
        dist = (xsq + xyneg) + csq
        out_ref[i] = jnp.argmin(dist, axis=0).astype(jnp.int32)[None, :]


def kernel(x, centroids):
    b, c, s = x.shape
    k = centroids.shape[1]
    out = pl.pallas_call(
        _labels_kernel,
        grid=(b // _BB,),
        in_specs=[
            pl.BlockSpec((_BB, c, s), lambda i: (i, 0, 0)),
            pl.BlockSpec((c, k), lambda i: (0, 0)),
        ],
        out_specs=pl.BlockSpec((_BB, 1, s), lambda i: (i, 0, 0)),
        out_shape=jax.ShapeDtypeStruct((b, 1, s), jnp.int32),
    )(x, centroids)
    return out.reshape(b, s)


# speedup vs baseline: 3.3969x; 1.0012x over previous
import functools

import jax
import jax.numpy as jnp
from jax.experimental import pallas as pl


_BB = 16


def _labels_kernel(x_ref, cent_ref, out_ref):
    cent = cent_ref[...]
    cneg = cent * -2.0
    csq = jnp.sum(cent * cent, axis=0)[:, None]
    for i in range(_BB):
        xb = x_ref[i]
        xyneg = jax.lax.dot_general(
            cneg, xb, (((0,), (0,)), ((), ())),
            preferred_element_type=jnp.float32)
        xsq = jnp.sum(xb * xb, axis=0, keepdims=True)
        dist = (xsq + xyneg) + csq
        out_ref[i] = jnp.argmin(dist, axis=0).astype(jnp.int32)[None, :]


def kernel(x, centroids):
    b, c, s = x.shape
    k = centroids.shape[1]
    out = pl.pallas_call(
        _labels_kernel,
        grid=(b // _BB,),
        in_specs=[
            pl.BlockSpec((_BB, c, s), lambda i: (i, 0, 0)),
            pl.BlockSpec((c, k), lambda i: (0, 0)),
        ],
        out_specs=pl.BlockSpec((_BB, 1, s), lambda i: (i, 0, 0)),
        out_shape=jax.ShapeDtypeStruct((b, 1, s), jnp.int32),
    )(x, centroids)
    return out.reshape(b, s)
